# baseline (speedup 1.0000x reference)
import jax
import jax.numpy as jnp
from jax.experimental import pallas as pl

_DTYPE_CODES = {
    jnp.dtype("int32"): 0,
    jnp.dtype("int64"): 1,
    jnp.dtype("uint32"): 2,
    jnp.dtype("uint64"): 3,
}


def _eq_kernel(codes_ref, out_ref):
    out_ref[...] = codes_ref[0, :] == codes_ref[1, :]


def kernel(x):
    n = x.size

    def _branch_perm():
        return jax.random.permutation(jax.random.key(0), n)

    observed = jax.eval_shape(_branch_perm).dtype
    expected = jax.dtypes.canonicalize_dtype(jnp.dtype("int64"))

    obs_code = _DTYPE_CODES[jnp.dtype(observed)]
    exp_code = _DTYPE_CODES[jnp.dtype(expected)]
    codes = jnp.array(
        [[obs_code, obs_code], [exp_code, exp_code]], dtype=jnp.int32
    )

    return pl.pallas_call(
        _eq_kernel,
        out_shape=jax.ShapeDtypeStruct((2,), jnp.bool_),
    )(codes)


# speedup vs baseline: 1.6010x; 1.6010x over previous
import jax
import jax.numpy as jnp
from jax.experimental import pallas as pl

_DTYPE_CODES = {
    jnp.dtype("int32"): 0,
    jnp.dtype("int64"): 1,
    jnp.dtype("uint32"): 2,
    jnp.dtype("uint64"): 3,
}


def _make_eq_kernel(obs_code: int, exp_code: int):
    def _eq_kernel(out_ref):
        observed = jnp.full((2,), obs_code, dtype=jnp.int32)
        expected = jnp.full((2,), exp_code, dtype=jnp.int32)
        out_ref[...] = observed == expected

    return _eq_kernel


def kernel(x):
    n = x.size

    def _branch_perm():
        return jax.random.permutation(jax.random.key(0), n)

    observed = jax.eval_shape(_branch_perm).dtype
    expected = jax.dtypes.canonicalize_dtype(jnp.dtype("int64"))

    obs_code = _DTYPE_CODES[jnp.dtype(observed)]
    exp_code = _DTYPE_CODES[jnp.dtype(expected)]
    return pl.pallas_call(
        _make_eq_kernel(obs_code, exp_code),
        out_shape=jax.ShapeDtypeStruct((2,), jnp.bool_),
    )()
